# baseline (speedup 1.0000x reference)
import functools

import jax
import jax.numpy as jnp
from jax import lax
from jax.experimental import pallas as pl
from jax.experimental.pallas import tpu as pltpu
from jax.experimental.pallas import tpu_sc as plsc

_B = 16
_L = 2048
_D = 256
_SEQ = 2 * _L + 1
_ROWS = _B * _SEQ
_HALF = _L // 2
_C = 32
_K = _HALF // _C
_NBUF = 7
_PIPE = 5


def _pipelined_chunks(chunk_specs):
    K = len(chunk_specs)
    gathers = {}
    scatters = {}

    def start_gathers(k):
        gathers[k] = [g(k % _NBUF) for g in chunk_specs[k][0]]

    def start_scatters(k):
        scatters[k] = [s(k % _NBUF) for s in chunk_specs[k][1]]

    for k in range(min(_PIPE, K)):
        start_gathers(k)
    waited = set()
    for k in range(K):
        for cp in gathers[k]:
            cp.wait()
        start_scatters(k)
        nk = k + _PIPE
        if nk < K:
            if nk >= _NBUF:
                for cp in scatters[nk - _NBUF]:
                    cp.wait()
                waited.add(nk - _NBUF)
            start_gathers(nk)
    for k in range(K):
        if k not in waited:
            for cp in scatters[k]:
                cp.wait()


def _iota16():
    return lax.iota(jnp.int32, 16)


def _sc_body(item_hbm, action_hbm, ctx_hbm, out_hbm, *refs):
    bufs_i = list(refs[0:_NBUF])
    bufs_a = list(refs[_NBUF:2 * _NBUF])
    idx_i = refs[2 * _NBUF]
    idx_a = refs[2 * _NBUF + 1]
    buf_c = refs[2 * _NBUF + 2]
    idx_c = refs[2 * _NBUF + 3]
    gsems = list(refs[2 * _NBUF + 4:2 * _NBUF + 4 + _NBUF])
    ssems = list(refs[2 * _NBUF + 4 + _NBUF:2 * _NBUF + 4 + 2 * _NBUF])
    csem = refs[2 * _NBUF + 4 + 2 * _NBUF]

    c = lax.axis_index("c")
    s = lax.axis_index("s")
    w = s * 2 + c
    b = w // 2
    h = w % 2
    s0 = b * _L + h * _HALF
    dbase = b * _SEQ + 1 + h * _L

    for k in range(_K):
        dst0 = dbase + 2 * k * _C
        for v in range(_C // 16):
            lanes = 2 * (16 * v + _iota16())
            idx_i[k, pl.ds(16 * v, 16)] = dst0 + lanes
            idx_a[k, pl.ds(16 * v, 16)] = dst0 + lanes + 1

    @pl.when(w == 0)
    def _():
        idx_c[pl.ds(0, 16)] = _SEQ * _iota16()
        cp = pltpu.make_async_copy(ctx_hbm, buf_c, csem)
        cp.start()
        cp.wait()
        cp2 = pltpu.make_async_copy(buf_c, out_hbm.at[idx_c], csem)
        cp2.start()
        cp2.wait()

    def gather(src, base, bufs, slot):
        cp = pltpu.make_async_copy(
            src.at[pl.ds(pl.multiple_of(base, 8), _C)], bufs[slot], gsems[slot])
        cp.start()
        return cp

    def scatter(bufs, idxs, k, slot):
        cp = pltpu.make_async_copy(
            bufs[slot], out_hbm.at[idxs.at[k]], ssems[slot])
        cp.start()
        return cp

    specs = []
    for k in range(_K):
        src0 = s0 + k * _C
        specs.append((
            [
                functools.partial(gather, item_hbm, src0, bufs_i),
                functools.partial(gather, action_hbm, src0, bufs_a),
            ],
            [
                functools.partial(scatter, bufs_i, idx_i, k),
                functools.partial(scatter, bufs_a, idx_a, k),
            ],
        ))
    _pipelined_chunks(specs)


def kernel(item_values, action_values, contextual_values):
    mesh = plsc.VectorSubcoreMesh(core_axis_name="c", subcore_axis_name="s")
    run = functools.partial(
        pl.kernel,
        out_type=jax.ShapeDtypeStruct((_ROWS, _D), jnp.float32),
        mesh=mesh,
        scratch_types=(
            [pltpu.VMEM((_C, _D), jnp.float32)] * (2 * _NBUF)
            + [pltpu.VMEM((_K, _C), jnp.int32)] * 2
            + [pltpu.VMEM((_B, _D), jnp.float32)]
            + [pltpu.VMEM((_B,), jnp.int32)]
            + [pltpu.SemaphoreType.DMA] * (2 * _NBUF + 1)
        ),
    )(_sc_body)
    out_values = run(item_values, action_values, contextual_values)
    out_lengths = jnp.full((_B,), _SEQ, dtype=jnp.int32)
    out_offsets = jnp.concatenate(
        [jnp.zeros((1,), jnp.int32), jnp.cumsum(out_lengths).astype(jnp.int32)]
    )
    return out_values, out_lengths, out_offsets
